# baseline (speedup 1.0000x reference)
import jax
import jax.numpy as jnp
from jax import lax
from jax.experimental import pallas as pl
from jax.experimental.pallas import tpu as pltpu
from jax.experimental.pallas import tpu_sc as plsc

S = 200
GN = 4096
D = 128
N_ARGS = 11
N_COMMANDS = 7
GROUP_VOCAB = 52
VOCAB_PAD = 64
ROWS = 5

NC, NS, L = 2, 16, 16
NW = NC * NS
C = 256
CHUNKS_PER_ROW = GN // C
N_CHUNKS = S * CHUNKS_PER_ROW
CHUNKS_PER_W = N_CHUNKS // NW
GSUB = 128


def _sc_relayout(args_ref, out_ref, inbuf, outbuf, idxbuf, sem):
    wid = lax.axis_index("s") * NC + lax.axis_index("c")
    iota = lax.broadcasted_iota(jnp.int32, (L,), 0)
    zero = jnp.zeros((L,), jnp.float32)

    def zrow(j, _):
        for k in range(N_ARGS, 16):
            outbuf[k, pl.ds(j * L, L)] = zero
        return 0
    lax.fori_loop(0, C // L, zrow, 0)

    def chunk(i, _):
        n = i * NW + wid
        s_idx = n // CHUNKS_PER_ROW
        g0 = (n % CHUNKS_PER_ROW) * C
        base = n * C

        pltpu.sync_copy(args_ref.at[pl.ds(base, C), :], inbuf)

        def xpose(j, _):
            rows = j * L + iota
            for k in range(N_ARGS):
                v = plsc.load_gather(inbuf, [rows, jnp.full((L,), k, jnp.int32)])
                outbuf[k, pl.ds(j * L, L)] = v
            return 0
        lax.fori_loop(0, C // L, xpose, 0)

        pltpu.sync_copy(outbuf, out_ref.at[s_idx, :, pl.ds(g0, C)])
        return 0

    lax.fori_loop(0, CHUNKS_PER_W, chunk, 0)


def _tc_body(cmd_ref, grp_ref, args_ref, w1_ref, w2_ref, b_ref, pos_ref, out_ref):
    iota = lax.broadcasted_iota(jnp.int32, (VOCAB_PAD, 1), 0)
    for r in range(ROWS):
        c = cmd_ref[r]
        g = grp_ref[r]
        oh_t = (iota == c).astype(jnp.float32) + (iota == g + N_COMMANDS).astype(jnp.float32)
        acc = lax.dot_general(
            oh_t, w1_ref[...], (((0,), (0,)), ((), ())),
            preferred_element_type=jnp.float32,
        )
        acc = acc + lax.dot_general(
            args_ref[r], w2_ref[...], (((0,), (0,)), ((), ())),
            preferred_element_type=jnp.float32,
        )
        pb = pos_ref[r] + b_ref[...]
        out_ref[r] = acc + pb


def kernel(commands, args, groups, command_embed, W_fcn, b_fcn, group_embed, pos_embed):
    args_flat = args.reshape(S * GN, N_ARGS)

    sc = pl.kernel(
        _sc_relayout,
        out_type=jax.ShapeDtypeStruct((S, 16, GN), jnp.float32),
        mesh=plsc.VectorSubcoreMesh(core_axis_name="c", subcore_axis_name="s"),
        scratch_types=[
            pltpu.VMEM((C, N_ARGS), jnp.float32),
            pltpu.VMEM((16, C), jnp.float32),
            pltpu.VMEM((C,), jnp.int32),
            pltpu.SemaphoreType.DMA,
        ],
        compiler_params=pltpu.CompilerParams(use_tc_tiling_on_sc=True,
                                             needs_layout_passes=False),
    )
    args_c = sc(args_flat)

    w1 = jnp.concatenate(
        [command_embed, group_embed,
         jnp.zeros((VOCAB_PAD - N_COMMANDS - GROUP_VOCAB, D), jnp.float32)], axis=0)
    w2 = jnp.concatenate([W_fcn.T, jnp.zeros((16 - N_ARGS, D), jnp.float32)], axis=0)
    b2 = b_fcn.reshape(1, D)
    cmd3 = commands.reshape(S, 1, GN).astype(jnp.int32)
    grp3 = groups.reshape(S, 1, GN).astype(jnp.int32)
    pos3 = pos_embed.reshape(-1, 1, D)

    grid = (S // ROWS,)
    out = pl.pallas_call(
        _tc_body,
        grid=grid,
        in_specs=[
            pl.BlockSpec((ROWS, 1, GN), lambda s: (s, 0, 0)),
            pl.BlockSpec((ROWS, 1, GN), lambda s: (s, 0, 0)),
            pl.BlockSpec((ROWS, 16, GN), lambda s: (s, 0, 0)),
            pl.BlockSpec((VOCAB_PAD, D), lambda s: (0, 0)),
            pl.BlockSpec((16, D), lambda s: (0, 0)),
            pl.BlockSpec((1, D), lambda s: (0, 0)),
            pl.BlockSpec((ROWS, 1, D), lambda s: (s, 0, 0)),
        ],
        out_specs=pl.BlockSpec((ROWS, GN, D), lambda s: (s, 0, 0)),
        out_shape=jax.ShapeDtypeStruct((S, GN, D), jnp.float32),
    )(cmd3, grp3, args_c, w1, w2, b2, pos3)
    return out


# speedup vs baseline: 1.4106x; 1.4106x over previous
import jax
import jax.numpy as jnp
from jax import lax
from jax.experimental import pallas as pl
from jax.experimental.pallas import tpu as pltpu
from jax.experimental.pallas import tpu_sc as plsc

S = 200
GN = 4096
D = 128
N_ARGS = 11
N_COMMANDS = 7
GROUP_VOCAB = 52
VOCAB_PAD = 64
ROWS = 5

NC, NS, L = 2, 16, 16
NW = NC * NS
C = 256
CHUNKS_PER_ROW = GN // C
N_CHUNKS = S * CHUNKS_PER_ROW
CHUNKS_PER_W = N_CHUNKS // NW
GSUB = 128


def _sc_relayout(args_ref, out_ref, inbuf, outbuf, sems):
    wid = lax.axis_index("s") * NC + lax.axis_index("c")
    iota = lax.broadcasted_iota(jnp.int32, (L,), 0)
    zero = jnp.zeros((L,), jnp.float32)

    def zrow(j, _):
        for p in range(2):
            for k in range(N_ARGS, 16):
                outbuf[p, k, pl.ds(j * L, L)] = zero
        return 0
    lax.fori_loop(0, C // L, zrow, 0)

    def in_copy(i, p):
        n = i * NW + wid
        return pltpu.make_async_copy(
            args_ref.at[pl.ds(n * C, C), :], inbuf.at[p], sems.at[p])

    def out_copy(i, p):
        n = i * NW + wid
        s_idx = n // CHUNKS_PER_ROW
        g0 = (n % CHUNKS_PER_ROW) * C
        return pltpu.make_async_copy(
            outbuf.at[p], out_ref.at[s_idx, :, pl.ds(g0, C)], sems.at[2 + p])

    def xpose(p):
        def body(j, _):
            rows = j * L + iota
            for k in range(N_ARGS):
                v = plsc.load_gather(inbuf.at[p], [rows, jnp.full((L,), k, jnp.int32)])
                outbuf[p, k, pl.ds(j * L, L)] = v
            return 0
        lax.fori_loop(0, C // L, body, 0)

    in_copy(0, 0).start()

    def pair(i, _):
        in_copy(2 * i + 1, 1).start()
        in_copy(2 * i, 0).wait()

        @pl.when(i > 0)
        def _():
            out_copy(2 * i - 2, 0).wait()
        xpose(0)
        out_copy(2 * i, 0).start()

        @pl.when(2 * i + 2 < CHUNKS_PER_W)
        def _():
            in_copy(2 * i + 2, 0).start()
        in_copy(2 * i + 1, 1).wait()

        @pl.when(i > 0)
        def _():
            out_copy(2 * i - 1, 1).wait()
        xpose(1)
        out_copy(2 * i + 1, 1).start()
        return 0

    lax.fori_loop(0, CHUNKS_PER_W // 2, pair, 0)
    out_copy(CHUNKS_PER_W - 2, 0).wait()
    out_copy(CHUNKS_PER_W - 1, 1).wait()


def _tc_body(cmd_ref, grp_ref, args_ref, w1_ref, w2_ref, b_ref, pos_ref, out_ref):
    iota = lax.broadcasted_iota(jnp.int32, (VOCAB_PAD, 1), 0)
    for r in range(ROWS):
        c = cmd_ref[r]
        g = grp_ref[r]
        oh_t = (iota == c).astype(jnp.float32) + (iota == g + N_COMMANDS).astype(jnp.float32)
        acc = lax.dot_general(
            oh_t, w1_ref[...], (((0,), (0,)), ((), ())),
            preferred_element_type=jnp.float32,
        )
        acc = acc + lax.dot_general(
            args_ref[r], w2_ref[...], (((0,), (0,)), ((), ())),
            preferred_element_type=jnp.float32,
        )
        pb = pos_ref[r] + b_ref[...]
        out_ref[r] = acc + pb


def kernel(commands, args, groups, command_embed, W_fcn, b_fcn, group_embed, pos_embed):
    args_flat = args.reshape(S * GN, N_ARGS)

    sc = pl.kernel(
        _sc_relayout,
        out_type=jax.ShapeDtypeStruct((S, 16, GN), jnp.float32),
        mesh=plsc.VectorSubcoreMesh(core_axis_name="c", subcore_axis_name="s"),
        scratch_types=[
            pltpu.VMEM((2, C, N_ARGS), jnp.float32),
            pltpu.VMEM((2, 16, C), jnp.float32),
            pltpu.SemaphoreType.DMA((4,)),
        ],
        compiler_params=pltpu.CompilerParams(use_tc_tiling_on_sc=True,
                                             needs_layout_passes=False),
    )
    args_c = sc(args_flat)

    w1 = jnp.concatenate(
        [command_embed, group_embed,
         jnp.zeros((VOCAB_PAD - N_COMMANDS - GROUP_VOCAB, D), jnp.float32)], axis=0)
    w2 = jnp.concatenate([W_fcn.T, jnp.zeros((16 - N_ARGS, D), jnp.float32)], axis=0)
    b2 = b_fcn.reshape(1, D)
    cmd3 = commands.reshape(S, 1, GN).astype(jnp.int32)
    grp3 = groups.reshape(S, 1, GN).astype(jnp.int32)
    pos3 = pos_embed.reshape(-1, 1, D)

    grid = (S // ROWS,)
    out = pl.pallas_call(
        _tc_body,
        grid=grid,
        in_specs=[
            pl.BlockSpec((ROWS, 1, GN), lambda s: (s, 0, 0)),
            pl.BlockSpec((ROWS, 1, GN), lambda s: (s, 0, 0)),
            pl.BlockSpec((ROWS, 16, GN), lambda s: (s, 0, 0)),
            pl.BlockSpec((VOCAB_PAD, D), lambda s: (0, 0)),
            pl.BlockSpec((16, D), lambda s: (0, 0)),
            pl.BlockSpec((1, D), lambda s: (0, 0)),
            pl.BlockSpec((ROWS, 1, D), lambda s: (s, 0, 0)),
        ],
        out_specs=pl.BlockSpec((ROWS, GN, D), lambda s: (s, 0, 0)),
        out_shape=jax.ShapeDtypeStruct((S, GN, D), jnp.float32),
    )(cmd3, grp3, args_c, w1, w2, b2, pos3)
    return out
